# initial kernel scaffold (unmeasured)
import jax
import jax.numpy as jnp
from jax import lax
from jax.experimental import pallas as pl
from jax.experimental.pallas import tpu as pltpu

N_DEV = 4


def kernel(x, router_W, route_idx, expert_W, shared_W):
    n_tok, d_model = x.shape
    e_loc, _, d_ff = expert_W.shape
    n_exp = router_W.shape[1]

    def body(x_ref, rw_ref, idx_ref, ew_ref, sw_ref, out_ref,
             comm_ref, send_sems, recv_sems):
        my = lax.axis_index("i")
        left = lax.rem(my + N_DEV - 1, N_DEV)
        right = lax.rem(my + 1, N_DEV)

        barrier_sem = pltpu.get_barrier_semaphore()
        for nbr in (left, right):
            pl.semaphore_signal(barrier_sem, inc=1, device_id=(nbr,),
                                device_id_type=pl.DeviceIdType.MESH)
        pl.semaphore_wait(barrier_sem, 2)

        hop = pltpu.make_async_remote_copy(
            src_ref=ew_ref, dst_ref=comm_ref.at[0],
            send_sem=send_sems.at[0], recv_sem=recv_sems.at[0],
            device_id=(right,), device_id_type=pl.DeviceIdType.MESH)
        hop.start()
        hops = [hop]

        xv = x_ref[:, :]
        scores = jnp.dot(xv, rw_ref[:, :], preferred_element_type=jnp.float32)
        s_max = jnp.max(scores, axis=1, keepdims=True)
        es = jnp.exp(scores - s_max)
        probs = es / jnp.sum(es, axis=1, keepdims=True)
        idx = idx_ref[:, :]
        eids = lax.broadcasted_iota(jnp.int32, (n_tok, n_exp), 1)
        p = jnp.sum(jnp.where(idx == eids, probs, 0.0), axis=1, keepdims=True)

        def add_shard(w_ref, src_dev):
            for el in range(e_loc):
                e = src_dev * e_loc + el
                scale = jnp.where(idx == e, p, 0.0)
                out_ref[:, :] += jnp.dot(
                    xv * scale, w_ref[el],
                    preferred_element_type=jnp.float32)

        out_ref[:, :] = jnp.dot(xv, sw_ref[:, :],
                                preferred_element_type=jnp.float32)
        add_shard(ew_ref, my)

        for h in range(N_DEV - 1):
            hops[h].wait_recv()
            if h < N_DEV - 2:
                nxt = pltpu.make_async_remote_copy(
                    src_ref=comm_ref.at[h], dst_ref=comm_ref.at[h + 1],
                    send_sem=send_sems.at[h + 1],
                    recv_sem=recv_sems.at[h + 1],
                    device_id=(right,), device_id_type=pl.DeviceIdType.MESH)
                nxt.start()
                hops.append(nxt)
            src_dev = lax.rem(my + N_DEV - 1 - h, N_DEV)
            add_shard(comm_ref.at[h], src_dev)

        for h in range(N_DEV - 1):
            hops[h].wait_send()

    return pl.pallas_call(
        body,
        out_shape=jax.ShapeDtypeStruct((n_tok, d_ff), jnp.float32),
        in_specs=[pl.BlockSpec(memory_space=pltpu.VMEM)] * 5,
        out_specs=pl.BlockSpec(memory_space=pltpu.VMEM),
        scratch_shapes=[
            pltpu.VMEM((N_DEV - 1, d_model, d_ff * e_loc // d_ff, ),
                       jnp.float32)
            if False else
            pltpu.VMEM((N_DEV - 1, e_loc, d_model, d_ff), jnp.float32),
            pltpu.SemaphoreType.DMA((N_DEV - 1,)),
            pltpu.SemaphoreType.DMA((N_DEV - 1,)),
        ],
        compiler_params=pltpu.CompilerParams(collective_id=0),
    )(x, router_W, route_idx, expert_W, shared_W)


# baseline (device time: 306801 ns/iter reference)
import jax
import jax.numpy as jnp
from jax import lax
from jax.experimental import pallas as pl
from jax.experimental.pallas import tpu as pltpu

N_DEV = 4


def kernel(x, router_W, route_idx, expert_W, shared_W):
    n_tok, d_model = x.shape
    e_loc, _, d_ff = expert_W.shape
    n_exp = router_W.shape[1]

    def body(x_ref, rw_ref, idx_ref, ew_ref, sw_ref, out_ref,
             comm_ref, send_sems, recv_sems):
        my = lax.axis_index("i")
        left = lax.rem(my + N_DEV - 1, N_DEV)
        right = lax.rem(my + 1, N_DEV)

        barrier_sem = pltpu.get_barrier_semaphore()
        for nbr in (left, right):
            pl.semaphore_signal(barrier_sem, inc=1, device_id=(nbr,),
                                device_id_type=pl.DeviceIdType.MESH)
        pl.semaphore_wait(barrier_sem, 2)

        hop = pltpu.make_async_remote_copy(
            src_ref=ew_ref, dst_ref=comm_ref.at[0],
            send_sem=send_sems.at[0], recv_sem=recv_sems.at[0],
            device_id=(right,), device_id_type=pl.DeviceIdType.MESH)
        hop.start()
        hops = [hop]

        xv = x_ref[:, :]
        scores = jnp.dot(xv, rw_ref[:, :], preferred_element_type=jnp.float32)
        s_max = jnp.max(scores, axis=1, keepdims=True)
        es = jnp.exp(scores - s_max)
        probs = es / jnp.sum(es, axis=1, keepdims=True)
        idx = idx_ref[:, :]
        eids = lax.broadcasted_iota(jnp.int32, (n_tok, n_exp), 1)
        p = jnp.sum(jnp.where(idx == eids, probs, 0.0), axis=1, keepdims=True)

        def add_shard(w_ref, src_dev):
            for el in range(e_loc):
                e = src_dev * e_loc + el
                scale = jnp.where(idx == e, p, 0.0)
                out_ref[:, :] += jnp.dot(
                    xv * scale, w_ref[el],
                    preferred_element_type=jnp.float32)

        out_ref[:, :] = jnp.dot(xv, sw_ref[:, :],
                                preferred_element_type=jnp.float32)
        add_shard(ew_ref, my)

        for h in range(N_DEV - 1):
            hops[h].wait_recv()
            if h < N_DEV - 2:
                nxt = pltpu.make_async_remote_copy(
                    src_ref=comm_ref.at[h], dst_ref=comm_ref.at[h + 1],
                    send_sem=send_sems.at[h + 1],
                    recv_sem=recv_sems.at[h + 1],
                    device_id=(right,), device_id_type=pl.DeviceIdType.MESH)
                nxt.start()
                hops.append(nxt)
            src_dev = lax.rem(my + N_DEV - 1 - h, N_DEV)
            add_shard(comm_ref.at[h], src_dev)

        for h in range(N_DEV - 1):
            hops[h].wait_send()

    return pl.pallas_call(
        body,
        out_shape=jax.ShapeDtypeStruct((n_tok, d_ff), jnp.float32),
        in_specs=[pl.BlockSpec(memory_space=pltpu.VMEM)] * 5,
        out_specs=pl.BlockSpec(memory_space=pltpu.VMEM),
        scratch_shapes=[
            pltpu.VMEM((N_DEV - 1, e_loc, d_model, d_ff), jnp.float32),
            pltpu.SemaphoreType.DMA((N_DEV - 1,)),
            pltpu.SemaphoreType.DMA((N_DEV - 1,)),
        ],
        compiler_params=pltpu.CompilerParams(
            collective_id=0, vmem_limit_bytes=60 * 1024 * 1024),
    )(x, router_W, route_idx, expert_W, shared_W)


# device time: 172066 ns/iter; 1.7830x vs baseline; 1.7830x over previous
import jax
import jax.numpy as jnp
from jax import lax
from jax.experimental import pallas as pl
from jax.experimental.pallas import tpu as pltpu

N_DEV = 4


def kernel(x, router_W, route_idx, expert_W, shared_W):
    n_tok, d_model = x.shape
    e_loc, _, d_ff = expert_W.shape
    n_exp = router_W.shape[1]

    def body(x_ref, rw_ref, idx_ref, ew_ref, sw_ref, out_ref,
             send_ref, comm_ref, send_sems, recv_sems):
        my = lax.axis_index("i")
        left = lax.rem(my + N_DEV - 1, N_DEV)
        right = lax.rem(my + 1, N_DEV)

        send_ref[:, :, :] = ew_ref[:, :, :].astype(jnp.bfloat16)

        barrier_sem = pltpu.get_barrier_semaphore()
        for nbr in (left, right):
            pl.semaphore_signal(barrier_sem, inc=1, device_id=(nbr,),
                                device_id_type=pl.DeviceIdType.MESH)
        pl.semaphore_wait(barrier_sem, 2)

        hop = pltpu.make_async_remote_copy(
            src_ref=send_ref, dst_ref=comm_ref.at[0],
            send_sem=send_sems.at[0], recv_sem=recv_sems.at[0],
            device_id=(right,), device_id_type=pl.DeviceIdType.MESH)
        hop.start()
        hops = [hop]

        xv = x_ref[:, :]
        scores = jnp.dot(xv, rw_ref[:, :], preferred_element_type=jnp.float32)
        s_max = jnp.max(scores, axis=1, keepdims=True)
        es = jnp.exp(scores - s_max)
        probs = es / jnp.sum(es, axis=1, keepdims=True)
        idx = idx_ref[:, :]
        eids = lax.broadcasted_iota(jnp.int32, (n_tok, n_exp), 1)
        p = jnp.sum(jnp.where(idx == eids, probs, 0.0), axis=1, keepdims=True)

        xb = xv.astype(jnp.bfloat16)

        def add_shard(w_ref, src_dev):
            for el in range(e_loc):
                e = src_dev * e_loc + el
                scale = jnp.where(idx == e, p, 0.0).astype(jnp.bfloat16)
                out_ref[:, :] += jnp.dot(
                    xb * scale, w_ref[el],
                    preferred_element_type=jnp.float32)

        out_ref[:, :] = jnp.dot(xv, sw_ref[:, :],
                                preferred_element_type=jnp.float32)
        add_shard(send_ref, my)

        for h in range(N_DEV - 1):
            hops[h].wait_recv()
            if h < N_DEV - 2:
                nxt = pltpu.make_async_remote_copy(
                    src_ref=comm_ref.at[h], dst_ref=comm_ref.at[h + 1],
                    send_sem=send_sems.at[h + 1],
                    recv_sem=recv_sems.at[h + 1],
                    device_id=(right,), device_id_type=pl.DeviceIdType.MESH)
                nxt.start()
                hops.append(nxt)
            src_dev = lax.rem(my + N_DEV - 1 - h, N_DEV)
            add_shard(comm_ref.at[h], src_dev)

        for h in range(N_DEV - 1):
            hops[h].wait_send()

    return pl.pallas_call(
        body,
        out_shape=jax.ShapeDtypeStruct((n_tok, d_ff), jnp.float32),
        in_specs=[pl.BlockSpec(memory_space=pltpu.VMEM)] * 5,
        out_specs=pl.BlockSpec(memory_space=pltpu.VMEM),
        scratch_shapes=[
            pltpu.VMEM((e_loc, d_model, d_ff), jnp.bfloat16),
            pltpu.VMEM((N_DEV - 1, e_loc, d_model, d_ff), jnp.bfloat16),
            pltpu.SemaphoreType.DMA((N_DEV - 1,)),
            pltpu.SemaphoreType.DMA((N_DEV - 1,)),
        ],
        compiler_params=pltpu.CompilerParams(
            collective_id=0, vmem_limit_bytes=60 * 1024 * 1024),
    )(x, router_W, route_idx, expert_W, shared_W)


# device time: 112146 ns/iter; 2.7357x vs baseline; 1.5343x over previous
import jax
import jax.numpy as jnp
from jax import lax
from jax.experimental import pallas as pl
from jax.experimental.pallas import tpu as pltpu

N_DEV = 4


def kernel(x, router_W, route_idx, expert_W, shared_W):
    n_tok, d_model = x.shape
    e_loc, _, d_ff = expert_W.shape
    n_exp = router_W.shape[1]
    e_half = e_loc // 2

    def body(x_ref, rw_ref, idx_ref, ew_ref, sw_ref, out_ref,
             send_ref, recv_l, recv_r, recv_dl, recv_dr,
             send_sems, recv_sems):
        my = lax.axis_index("i")
        left = lax.rem(my + N_DEV - 1, N_DEV)
        right = lax.rem(my + 1, N_DEV)
        diag = lax.rem(my + 2, N_DEV)

        send_ref[:, :, :] = ew_ref[:, :, :].astype(jnp.bfloat16)

        barrier_sem = pltpu.get_barrier_semaphore()
        for nbr in (left, right):
            pl.semaphore_signal(barrier_sem, inc=1, device_id=(nbr,),
                                device_id_type=pl.DeviceIdType.MESH)
        pl.semaphore_wait(barrier_sem, 2)

        cw1 = pltpu.make_async_remote_copy(
            src_ref=send_ref, dst_ref=recv_l,
            send_sem=send_sems.at[0], recv_sem=recv_sems.at[0],
            device_id=(right,), device_id_type=pl.DeviceIdType.MESH)
        ccw1 = pltpu.make_async_remote_copy(
            src_ref=send_ref, dst_ref=recv_r,
            send_sem=send_sems.at[1], recv_sem=recv_sems.at[1],
            device_id=(left,), device_id_type=pl.DeviceIdType.MESH)
        cw1.start()
        ccw1.start()

        xv = x_ref[:, :]
        scores = jnp.dot(xv, rw_ref[:, :], preferred_element_type=jnp.float32)
        s_max = jnp.max(scores, axis=1, keepdims=True)
        es = jnp.exp(scores - s_max)
        probs = es / jnp.sum(es, axis=1, keepdims=True)
        idx = idx_ref[:, :]
        eids = lax.broadcasted_iota(jnp.int32, (n_tok, n_exp), 1)
        p = jnp.sum(jnp.where(idx == eids, probs, 0.0), axis=1, keepdims=True)

        xb = xv.astype(jnp.bfloat16)

        def masked_x(src_dev, lo, n):
            blocks = []
            for el in range(n):
                e = src_dev * e_loc + lo + el
                scale = jnp.where(idx == e, p, 0.0).astype(jnp.bfloat16)
                blocks.append(xb * scale)
            return jnp.concatenate(blocks, axis=1)

        a_loc = jnp.concatenate(
            [masked_x(my, 0, e_loc), xb], axis=1)
        b_loc = jnp.concatenate(
            [send_ref[:, :, :].reshape(e_loc * d_model, d_ff),
             sw_ref[:, :].astype(jnp.bfloat16)], axis=0)
        out_ref[:, :] = jnp.dot(a_loc, b_loc,
                                preferred_element_type=jnp.float32)

        cw1.wait_recv()
        cw2 = pltpu.make_async_remote_copy(
            src_ref=recv_l.at[pl.ds(0, e_half)], dst_ref=recv_dl,
            send_sem=send_sems.at[2], recv_sem=recv_sems.at[2],
            device_id=(right,), device_id_type=pl.DeviceIdType.MESH)
        cw2.start()
        out_ref[:, :] += jnp.dot(
            masked_x(left, 0, e_loc),
            recv_l[:, :, :].reshape(e_loc * d_model, d_ff),
            preferred_element_type=jnp.float32)

        ccw1.wait_recv()
        ccw2 = pltpu.make_async_remote_copy(
            src_ref=recv_r.at[pl.ds(e_half, e_half)], dst_ref=recv_dr,
            send_sem=send_sems.at[3], recv_sem=recv_sems.at[3],
            device_id=(left,), device_id_type=pl.DeviceIdType.MESH)
        ccw2.start()
        out_ref[:, :] += jnp.dot(
            masked_x(right, 0, e_loc),
            recv_r[:, :, :].reshape(e_loc * d_model, d_ff),
            preferred_element_type=jnp.float32)

        cw2.wait_recv()
        ccw2.wait_recv()
        b_diag = jnp.concatenate(
            [recv_dl[:, :, :].reshape(e_half * d_model, d_ff),
             recv_dr[:, :, :].reshape(e_half * d_model, d_ff)], axis=0)
        out_ref[:, :] += jnp.dot(
            masked_x(diag, 0, e_loc), b_diag,
            preferred_element_type=jnp.float32)

        for r in (cw1, ccw1, cw2, ccw2):
            r.wait_send()

    return pl.pallas_call(
        body,
        out_shape=jax.ShapeDtypeStruct((n_tok, d_ff), jnp.float32),
        in_specs=[pl.BlockSpec(memory_space=pltpu.VMEM)] * 5,
        out_specs=pl.BlockSpec(memory_space=pltpu.VMEM),
        scratch_shapes=[
            pltpu.VMEM((e_loc, d_model, d_ff), jnp.bfloat16),
            pltpu.VMEM((e_loc, d_model, d_ff), jnp.bfloat16),
            pltpu.VMEM((e_loc, d_model, d_ff), jnp.bfloat16),
            pltpu.VMEM((e_half, d_model, d_ff), jnp.bfloat16),
            pltpu.VMEM((e_half, d_model, d_ff), jnp.bfloat16),
            pltpu.SemaphoreType.DMA((4,)),
            pltpu.SemaphoreType.DMA((4,)),
        ],
        compiler_params=pltpu.CompilerParams(
            collective_id=0, vmem_limit_bytes=60 * 1024 * 1024),
    )(x, router_W, route_idx, expert_W, shared_W)


# device time: 107688 ns/iter; 2.8490x vs baseline; 1.0414x over previous
import jax
import jax.numpy as jnp
from jax import lax
from jax.experimental import pallas as pl
from jax.experimental.pallas import tpu as pltpu

N_DEV = 4


def kernel(x, router_W, route_idx, expert_W, shared_W):
    n_tok, d_model = x.shape
    e_loc, _, d_ff = expert_W.shape
    n_exp = router_W.shape[1]
    e_half = e_loc // 2
    k_half = e_half * d_model

    def body(x_ref, rw_ref, idx_ref, ew_ref, sw_ref, out_ref,
             send_ref, recv_l, recv_r, recv_dl, recv_dr,
             send_sems, recv_sems):
        my = lax.axis_index("i")
        left = lax.rem(my + N_DEV - 1, N_DEV)
        right = lax.rem(my + 1, N_DEV)
        diag = lax.rem(my + 2, N_DEV)

        send_ref[:, :, :] = ew_ref[:, :, :].astype(jnp.bfloat16)

        barrier_sem = pltpu.get_barrier_semaphore()
        for nbr in (left, right):
            pl.semaphore_signal(barrier_sem, inc=1, device_id=(nbr,),
                                device_id_type=pl.DeviceIdType.MESH)
        pl.semaphore_wait(barrier_sem, 2)

        def copy(src, dst, sem, dev):
            return pltpu.make_async_remote_copy(
                src_ref=src, dst_ref=dst,
                send_sem=send_sems.at[sem], recv_sem=recv_sems.at[sem],
                device_id=(dev,), device_id_type=pl.DeviceIdType.MESH)

        lo_half = pl.ds(0, e_half)
        hi_half = pl.ds(e_half, e_half)

        cw_h1 = copy(send_ref.at[lo_half], recv_l.at[lo_half], 0, right)
        cw_h2 = copy(send_ref.at[hi_half], recv_l.at[hi_half], 1, right)
        ccw_h1 = copy(send_ref.at[lo_half], recv_r.at[lo_half], 2, left)
        ccw_h2 = copy(send_ref.at[hi_half], recv_r.at[hi_half], 3, left)
        for c in (cw_h1, cw_h2, ccw_h1, ccw_h2):
            c.start()

        xv = x_ref[:, :]
        scores = jnp.dot(xv, rw_ref[:, :], preferred_element_type=jnp.float32)
        s_max = jnp.max(scores, axis=1, keepdims=True)
        es = jnp.exp(scores - s_max)
        probs = es / jnp.sum(es, axis=1, keepdims=True)
        idx = idx_ref[:, :]
        eids = lax.broadcasted_iota(jnp.int32, (n_tok, n_exp), 1)
        p = jnp.sum(jnp.where(idx == eids, probs, 0.0), axis=1, keepdims=True)

        xb = xv.astype(jnp.bfloat16)

        def a_blocks(src_dev, lo, n):
            blocks = []
            for el in range(n):
                e = src_dev * e_loc + lo + el
                scale = jnp.where(idx == e, p, 0.0).astype(jnp.bfloat16)
                blocks.append(xb * scale)
            return jnp.concatenate(blocks, axis=1)

        a_loc = jnp.concatenate([a_blocks(my, 0, e_loc), xb], axis=1)
        b_loc = jnp.concatenate(
            [send_ref[:, :, :].reshape(e_loc * d_model, d_ff),
             sw_ref[:, :].astype(jnp.bfloat16)], axis=0)
        out_ref[:, :] = jnp.dot(a_loc, b_loc,
                                preferred_element_type=jnp.float32)

        a = a_blocks(left, 0, e_half)
        cw_h1.wait_recv()
        cw_fwd = copy(recv_l.at[lo_half], recv_dl, 4, right)
        cw_fwd.start()
        out_ref[:, :] += jnp.dot(
            a, recv_l[0:e_half].reshape(k_half, d_ff),
            preferred_element_type=jnp.float32)

        a = a_blocks(right, 0, e_half)
        ccw_h1.wait_recv()
        out_ref[:, :] += jnp.dot(
            a, recv_r[0:e_half].reshape(k_half, d_ff),
            preferred_element_type=jnp.float32)

        a = a_blocks(left, e_half, e_half)
        cw_h2.wait_recv()
        out_ref[:, :] += jnp.dot(
            a, recv_l[e_half:e_loc].reshape(k_half, d_ff),
            preferred_element_type=jnp.float32)

        a = a_blocks(right, e_half, e_half)
        ccw_h2.wait_recv()
        ccw_fwd = copy(recv_r.at[hi_half], recv_dr, 5, left)
        ccw_fwd.start()
        out_ref[:, :] += jnp.dot(
            a, recv_r[e_half:e_loc].reshape(k_half, d_ff),
            preferred_element_type=jnp.float32)

        a = a_blocks(diag, 0, e_loc)
        cw_fwd.wait_recv()
        ccw_fwd.wait_recv()
        b_diag = jnp.concatenate(
            [recv_dl[:, :, :].reshape(k_half, d_ff),
             recv_dr[:, :, :].reshape(k_half, d_ff)], axis=0)
        out_ref[:, :] += jnp.dot(a, b_diag,
                                 preferred_element_type=jnp.float32)

        for c in (cw_h1, cw_h2, ccw_h1, ccw_h2, cw_fwd, ccw_fwd):
            c.wait_send()

    return pl.pallas_call(
        body,
        out_shape=jax.ShapeDtypeStruct((n_tok, d_ff), jnp.float32),
        in_specs=[pl.BlockSpec(memory_space=pltpu.VMEM)] * 5,
        out_specs=pl.BlockSpec(memory_space=pltpu.VMEM),
        scratch_shapes=[
            pltpu.VMEM((e_loc, d_model, d_ff), jnp.bfloat16),
            pltpu.VMEM((e_loc, d_model, d_ff), jnp.bfloat16),
            pltpu.VMEM((e_loc, d_model, d_ff), jnp.bfloat16),
            pltpu.VMEM((e_half, d_model, d_ff), jnp.bfloat16),
            pltpu.VMEM((e_half, d_model, d_ff), jnp.bfloat16),
            pltpu.SemaphoreType.DMA((6,)),
            pltpu.SemaphoreType.DMA((6,)),
        ],
        compiler_params=pltpu.CompilerParams(
            collective_id=0, vmem_limit_bytes=60 * 1024 * 1024),
    )(x, router_W, route_idx, expert_W, shared_W)


# device time: 107068 ns/iter; 2.8655x vs baseline; 1.0058x over previous
import jax
import jax.numpy as jnp
from jax import lax
from jax.experimental import pallas as pl
from jax.experimental.pallas import tpu as pltpu

N_DEV = 4


def kernel(x, router_W, route_idx, expert_W, shared_W):
    n_tok, d_model = x.shape
    e_loc, _, d_ff = expert_W.shape
    n_exp = router_W.shape[1]
    e_half = e_loc // 2
    k_half = e_half * d_model

    def body(x_ref, rw_ref, idx_ref, ew_ref, sw_ref, out_ref,
             send_ref, recv_l, recv_r, recv_dl, recv_dr,
             send_sems, recv_sems):
        my = lax.axis_index("i")
        left = lax.rem(my + N_DEV - 1, N_DEV)
        right = lax.rem(my + 1, N_DEV)
        diag = lax.rem(my + 2, N_DEV)

        barrier_sem = pltpu.get_barrier_semaphore()
        for nbr in (left, right):
            pl.semaphore_signal(barrier_sem, inc=1, device_id=(nbr,),
                                device_id_type=pl.DeviceIdType.MESH)

        def copy(src, dst, sem, dev):
            return pltpu.make_async_remote_copy(
                src_ref=src, dst_ref=dst,
                send_sem=send_sems.at[sem], recv_sem=recv_sems.at[sem],
                device_id=(dev,), device_id_type=pl.DeviceIdType.MESH)

        lo_half = pl.ds(0, e_half)
        hi_half = pl.ds(e_half, e_half)

        send_ref[0:e_half] = ew_ref[0:e_half].astype(jnp.bfloat16)
        pl.semaphore_wait(barrier_sem, 2)

        cw_h1 = copy(send_ref.at[lo_half], recv_l.at[lo_half], 0, right)
        ccw_h1 = copy(send_ref.at[lo_half], recv_r.at[lo_half], 2, left)
        cw_h1.start()
        ccw_h1.start()

        send_ref[e_half:e_loc] = ew_ref[e_half:e_loc].astype(jnp.bfloat16)
        cw_h2 = copy(send_ref.at[hi_half], recv_l.at[hi_half], 1, right)
        ccw_h2 = copy(send_ref.at[hi_half], recv_r.at[hi_half], 3, left)
        cw_h2.start()
        ccw_h2.start()

        xv = x_ref[:, :]
        scores = jnp.dot(xv, rw_ref[:, :], preferred_element_type=jnp.float32)
        s_max = jnp.max(scores, axis=1, keepdims=True)
        es = jnp.exp(scores - s_max)
        probs = es / jnp.sum(es, axis=1, keepdims=True)
        idx = idx_ref[:, :]
        eids = lax.broadcasted_iota(jnp.int32, (n_tok, n_exp), 1)
        p = jnp.sum(jnp.where(idx == eids, probs, 0.0), axis=1, keepdims=True)

        xb = xv.astype(jnp.bfloat16)

        def a_blocks(src_dev, lo, n):
            blocks = []
            for el in range(n):
                e = src_dev * e_loc + lo + el
                scale = jnp.where(idx == e, p, 0.0).astype(jnp.bfloat16)
                blocks.append(xb * scale)
            return jnp.concatenate(blocks, axis=1)

        a_loc = jnp.concatenate([a_blocks(my, 0, e_loc), xb], axis=1)
        b_loc = jnp.concatenate(
            [send_ref[:, :, :].reshape(e_loc * d_model, d_ff),
             sw_ref[:, :].astype(jnp.bfloat16)], axis=0)
        out_ref[:, :] = jnp.dot(a_loc, b_loc,
                                preferred_element_type=jnp.float32)

        a = a_blocks(left, 0, e_half)
        cw_h1.wait_recv()
        cw_fwd = copy(recv_l.at[lo_half], recv_dl, 4, right)
        cw_fwd.start()
        out_ref[:, :] += jnp.dot(
            a, recv_l[0:e_half].reshape(k_half, d_ff),
            preferred_element_type=jnp.float32)

        a = a_blocks(right, 0, e_half)
        ccw_h1.wait_recv()
        out_ref[:, :] += jnp.dot(
            a, recv_r[0:e_half].reshape(k_half, d_ff),
            preferred_element_type=jnp.float32)

        a = a_blocks(left, e_half, e_half)
        cw_h2.wait_recv()
        out_ref[:, :] += jnp.dot(
            a, recv_l[e_half:e_loc].reshape(k_half, d_ff),
            preferred_element_type=jnp.float32)

        a = a_blocks(right, e_half, e_half)
        ccw_h2.wait_recv()
        ccw_fwd = copy(recv_r.at[hi_half], recv_dr, 5, left)
        ccw_fwd.start()
        out_ref[:, :] += jnp.dot(
            a, recv_r[e_half:e_loc].reshape(k_half, d_ff),
            preferred_element_type=jnp.float32)

        a = a_blocks(diag, 0, e_half)
        cw_fwd.wait_recv()
        out_ref[:, :] += jnp.dot(
            a, recv_dl[:, :, :].reshape(k_half, d_ff),
            preferred_element_type=jnp.float32)

        a = a_blocks(diag, e_half, e_half)
        ccw_fwd.wait_recv()
        out_ref[:, :] += jnp.dot(
            a, recv_dr[:, :, :].reshape(k_half, d_ff),
            preferred_element_type=jnp.float32)

        for c in (cw_h1, cw_h2, ccw_h1, ccw_h2, cw_fwd, ccw_fwd):
            c.wait_send()

    return pl.pallas_call(
        body,
        out_shape=jax.ShapeDtypeStruct((n_tok, d_ff), jnp.float32),
        in_specs=[pl.BlockSpec(memory_space=pltpu.VMEM)] * 5,
        out_specs=pl.BlockSpec(memory_space=pltpu.VMEM),
        scratch_shapes=[
            pltpu.VMEM((e_loc, d_model, d_ff), jnp.bfloat16),
            pltpu.VMEM((e_loc, d_model, d_ff), jnp.bfloat16),
            pltpu.VMEM((e_loc, d_model, d_ff), jnp.bfloat16),
            pltpu.VMEM((e_half, d_model, d_ff), jnp.bfloat16),
            pltpu.VMEM((e_half, d_model, d_ff), jnp.bfloat16),
            pltpu.SemaphoreType.DMA((6,)),
            pltpu.SemaphoreType.DMA((6,)),
        ],
        compiler_params=pltpu.CompilerParams(
            collective_id=0, vmem_limit_bytes=60 * 1024 * 1024),
    )(x, router_W, route_idx, expert_W, shared_W)
